# baseline (speedup 1.0000x reference)
import functools

import jax
import jax.numpy as jnp
from jax import lax
from jax.experimental import pallas as pl
from jax.experimental.pallas import tpu as pltpu
from jax.experimental.pallas import tpu_sc as plsc

BATCH = 1024
NT = 200
D = 128
ROWS = BATCH * NT
NC, NS, L = 2, 16, 16
NW = NC * NS
B_PER_W = ROWS // NW
CHUNK = 128
NCHUNK = B_PER_W // CHUNK
VECS = CHUNK // L
NBUF = 5
ROUNDS = NCHUNK // NBUF
TBL_ROWS = 158


def _sc_gather(idx_hbm, seq_hbm, table_hbm):
    mesh = plsc.VectorSubcoreMesh(core_axis_name="c", subcore_axis_name="s")

    @functools.partial(
        pl.kernel,
        out_type=jax.ShapeDtypeStruct((ROWS, D), jnp.float32),
        mesh=mesh,
        scratch_types=[
            pltpu.VMEM((NCHUNK, CHUNK), jnp.int32),
            pltpu.VMEM((NBUF, CHUNK, D), jnp.float32),
            pltpu.VMEM((L,), jnp.int32),
            pltpu.VMEM_SHARED((TBL_ROWS, D), jnp.float32),
            pltpu.SemaphoreType.DMA((NBUF,)),
            pltpu.SemaphoreType.DMA((NBUF,)),
        ],
    )
    def body(idx_ref, seq_ref, tbl_ref, out_ref,
             idx_v, rows_v, seq_v, tbl_sh, gsem, psem):
        sid = lax.axis_index("s")
        wid = sid * NC + lax.axis_index("c")
        base = wid * B_PER_W

        @pl.when(sid == 0)
        def _():
            pltpu.sync_copy(tbl_ref, tbl_sh)

        pltpu.sync_copy(idx_ref.at[wid], idx_v)
        pltpu.sync_copy(seq_ref, seq_v)
        seq = seq_v[...]
        lane = lax.iota(jnp.int32, L)

        def xform(j):
            for k in range(VECS):
                pos0 = base + j * CHUNK + k * L
                t = lax.rem(pos0 + lane, NT)
                v = idx_v[j, pl.ds(k * L, L)]
                idx_v[j, pl.ds(k * L, L)] = jnp.where(t < seq, v + 1, 0)

        for j in range(NBUF):
            xform(j)
        plsc.subcore_barrier()

        def gather_cp(b, j):
            return pltpu.make_async_copy(
                tbl_sh.at[idx_v.at[j]], rows_v.at[b], gsem.at[b])

        def put_cp(b, j):
            return pltpu.make_async_copy(
                rows_v.at[b], out_ref.at[pl.ds(base + j * CHUNK, CHUNK)],
                psem.at[b])

        for b in range(NBUF):
            gather_cp(b, b).start()

        def round_body(r, _):
            for b in range(NBUF):
                j = r * NBUF + b
                gather_cp(b, j).wait()
                put_cp(b, j).start()
                jn = j + NBUF

                @pl.when(jn < NCHUNK)
                def _():
                    xform(jn)
                    put_cp(b, j).wait()
                    gather_cp(b, jn).start()
            return 0

        lax.fori_loop(0, ROUNDS, round_body, 0)

        for b in range(NBUF):
            put_cp(b, (ROUNDS - 1) * NBUF + b).wait()

    return body(idx_hbm, seq_hbm, table_hbm)


def kernel(text, seq_len, text_embed, text_embed_ko):
    del text_embed
    idx = text.reshape(NW, NCHUNK, CHUNK).astype(jnp.int32)
    seq = jnp.full((L,), seq_len, dtype=jnp.int32)
    out = _sc_gather(idx, seq, text_embed_ko)
    return out.reshape(BATCH, NT, D)
